# baseline (speedup 1.0000x reference)
import functools

import jax
import jax.numpy as jnp
from jax import lax
from jax.experimental import pallas as pl
from jax.experimental.pallas import tpu as pltpu
from jax.experimental.pallas import tpu_sc as plsc

B, S, D = 4, 2048, 768
E, TOP_K = 8, 2
FFN = D * 4
T = B * S
T2 = T * TOP_K

BM_R = 1024
BM = 128
PADDED = T2 + E * BM
NBLK = PADDED // BM
NBE = 144

NC, NS = 2, 16
NW = NC * NS
GROWS = PADDED // NW
GCH = 64
CROWS = T // NW
CCH = 32

_sc_mesh = plsc.VectorSubcoreMesh(core_axis_name="c", subcore_axis_name="s")


def _splat(s, dtype=jnp.int32):
    return jnp.full((16,), s, dtype)


_sc_params = pltpu.CompilerParams(needs_layout_passes=False)


def _router_body(x_ref, wg_ref, logits_ref, sel_ref, wtk_ref):
    x = x_ref[...]
    logits = lax.dot_general(x, wg_ref[...], (((1,), (1,)), ((), ())),
                             preferred_element_type=jnp.float32)
    logits_ref[...] = logits
    iota = lax.broadcasted_iota(jnp.int32, (BM_R, E), 1)
    m1 = jnp.max(logits, axis=1, keepdims=True)
    i1 = jnp.min(jnp.where(logits == m1, iota, E), axis=1, keepdims=True)
    l2 = jnp.where(iota == i1, -jnp.inf, logits)
    m2 = jnp.max(l2, axis=1, keepdims=True)
    i2 = jnp.min(jnp.where(l2 == m2, iota, E), axis=1, keepdims=True)
    w1 = 1.0 / (1.0 + jnp.exp(m2 - m1))
    sel_ref[...] = jnp.concatenate([i1, i2], axis=1)
    wtk_ref[...] = jnp.concatenate([w1, 1.0 - w1], axis=1)


def _router(hs, W_gate):
    return pl.pallas_call(
        _router_body,
        grid=(T // BM_R,),
        in_specs=[
            pl.BlockSpec((BM_R, D), lambda i: (i, 0)),
            pl.BlockSpec((E, D), lambda i: (0, 0)),
        ],
        out_specs=[
            pl.BlockSpec((BM_R, E), lambda i: (i, 0)),
            pl.BlockSpec((BM_R, TOP_K), lambda i: (i, 0)),
            pl.BlockSpec((BM_R, TOP_K), lambda i: (i, 0)),
        ],
        out_shape=[
            jax.ShapeDtypeStruct((T, E), jnp.float32),
            jax.ShapeDtypeStruct((T, TOP_K), jnp.int32),
            jax.ShapeDtypeStruct((T, TOP_K), jnp.float32),
        ],
    )(hs, W_gate)


@functools.partial(
    pl.kernel,
    out_type=[
        jax.ShapeDtypeStruct((PADDED,), jnp.int32),
        jax.ShapeDtypeStruct((PADDED,), jnp.float32),
        jax.ShapeDtypeStruct((T2,), jnp.int32),
        jax.ShapeDtypeStruct((NBE,), jnp.int32),
    ],
    mesh=_sc_mesh,
    compiler_params=_sc_params,
    name="sc_sort",
    scratch_types=[
        pltpu.VMEM((T2,), jnp.int32),
        pltpu.VMEM((T2,), jnp.float32),
        pltpu.VMEM((PADDED,), jnp.int32),
        pltpu.VMEM((PADDED,), jnp.float32),
        pltpu.VMEM((T2,), jnp.int32),
        pltpu.VMEM((NBE,), jnp.int32),
    ],
)
def _sc_sort(sel_hbm, w_hbm, tok_hbm, ws_hbm, pos_hbm, be_hbm,
             sel_v, w_v, tok_v, ws_v, pos_v, be_v):
    wid = lax.axis_index("s") * NC + lax.axis_index("c")

    @pl.when(wid == 0)
    def _():
        pltpu.sync_copy(sel_hbm, sel_v)
        pltpu.sync_copy(w_hbm, w_v)
        zi = jnp.zeros((16,), jnp.int32)
        zf = jnp.zeros((16,), jnp.float32)

        def init(i, c):
            tok_v[pl.ds(i * 16, 16)] = zi
            ws_v[pl.ds(i * 16, 16)] = zf
            return c
        lax.fori_loop(0, PADDED // 16, init, 0)

        def hist(i, cnts):
            ev = sel_v[pl.ds(i * 16, 16)]
            return tuple(cnts[e] + jnp.sum((ev == _splat(e)).astype(jnp.int32))
                         for e in range(E))
        cnts = lax.fori_loop(0, T2 // 16, hist, (jnp.int32(0),) * E)

        starts, ends = [], []
        s = jnp.int32(0)
        for e in range(E):
            starts.append(s)
            s = s + ((cnts[e] + BM - 1) // BM) * BM
            ends.append(s)
        total = s

        iota16 = lax.iota(jnp.int32, 16)
        for vb in range(NBE // 16):
            bstart = (iota16 + _splat(vb * 16)) * _splat(BM)
            acc = jnp.zeros((16,), jnp.int32)
            for e in range(E):
                acc = acc + (bstart >= _splat(ends[e])).astype(jnp.int32)
            be_v[pl.ds(vb * 16, 16)] = jnp.where(bstart < _splat(total), acc,
                                                 _splat(-1))

        def scat(i, offs):
            ds = pl.ds(i * 16, 16)
            jv = _splat(i * 16) + iota16
            ev = sel_v[ds]
            wv = w_v[ds]
            tv = jv >> _splat(1)
            kv = jv & _splat(1)
            posidx = kv * _splat(T) + tv
            p = jnp.zeros((16,), jnp.int32)
            new_offs = []
            for e in range(E):
                me = ev == _splat(e)
                mi = me.astype(jnp.int32)
                cs = jnp.cumsum(mi)
                p = jnp.where(me, _splat(offs[e]) + cs - _splat(1), p)
                new_offs.append(offs[e] + jnp.sum(mi))
            plsc.store_scatter(tok_v, [p], tv)
            plsc.store_scatter(ws_v, [p], wv)
            plsc.store_scatter(pos_v, [posidx], p)
            return tuple(new_offs)
        lax.fori_loop(0, T2 // 16, scat, tuple(starts))

        pltpu.sync_copy(tok_v, tok_hbm)
        pltpu.sync_copy(ws_v, ws_hbm)
        pltpu.sync_copy(pos_v, pos_hbm)
        pltpu.sync_copy(be_v, be_hbm)


GCH2 = 32


@functools.partial(
    pl.kernel,
    out_type=jax.ShapeDtypeStruct((PADDED, D), jnp.float32),
    mesh=_sc_mesh,
    compiler_params=_sc_params,
    name="sc_scatter_x",
    scratch_types=[
        pltpu.VMEM((GCH2,), jnp.int32),
        pltpu.VMEM((GCH2,), jnp.int32),
        pltpu.VMEM((GCH2,), jnp.int32),
        pltpu.VMEM((GCH2,), jnp.int32),
        pltpu.VMEM((GCH2, D), jnp.float32),
        pltpu.VMEM((GCH2, D), jnp.float32),
        pltpu.SemaphoreType.DMA,
        pltpu.SemaphoreType.DMA,
        pltpu.SemaphoreType.DMA,
        pltpu.SemaphoreType.DMA,
    ],
)
def _sc_scatter_x(hs_hbm, pos_hbm, xs_hbm,
                  i0a, i1a, i0b, i1b, rows0, rows1, sr0, sr1, sw0, sw1):
    wid = lax.axis_index("s") * NC + lax.axis_index("c")
    base = wid * CROWS
    nch = CROWS // GCH2
    bufs, i0s, i1s = (rows0, rows1), (i0a, i0b), (i1a, i1b)
    rsems, wsems = (sr0, sr1), (sw0, sw1)
    gets, putsA, putsB = [None] * nch, [None] * nch, [None] * nch

    def fire(c):
        gets[c] = pltpu.async_copy(
            hs_hbm.at[pl.ds(base + c * GCH2, GCH2)], bufs[c % 2], rsems[c % 2])

    fire(0)
    for c in range(nch):
        b = c % 2
        if c + 1 < nch:
            if c >= 1:
                putsA[c - 1].wait()
                putsB[c - 1].wait()
            fire(c + 1)
        off = base + c * GCH2
        pltpu.sync_copy(pos_hbm.at[pl.ds(off, GCH2)], i0s[b])
        pltpu.sync_copy(pos_hbm.at[pl.ds(T + off, GCH2)], i1s[b])
        gets[c].wait()
        putsA[c] = pltpu.async_copy(bufs[b], xs_hbm.at[i0s[b]], wsems[b])
        putsB[c] = pltpu.async_copy(bufs[b], xs_hbm.at[i1s[b]], wsems[b])
    if nch >= 2:
        putsA[nch - 2].wait()
        putsB[nch - 2].wait()
    putsA[nch - 1].wait()
    putsB[nch - 1].wait()


def _moe_body(be_ref, x_ref, w1_ref, wg_ref, w2_ref, wt_ref, out_ref):
    i = pl.program_id(0)

    @pl.when(be_ref[i] >= 0)
    def _():
        x = x_ref[...].astype(jnp.bfloat16)
        a = lax.dot_general(x, w1_ref[0], (((1,), (1,)), ((), ())),
                            preferred_element_type=jnp.float32)
        g = lax.dot_general(x, wg_ref[0], (((1,), (1,)), ((), ())),
                            preferred_element_type=jnp.float32)
        h = ((a * jax.nn.sigmoid(a)) * g).astype(jnp.bfloat16)
        o = lax.dot_general(h, w2_ref[0], (((1,), (1,)), ((), ())),
                            preferred_element_type=jnp.float32)
        out_ref[...] = o * wt_ref[0, 0][:, None]


def _grouped_mlp(be, xs, W1, Wg, W2, ws):
    wsr = ws.reshape(NBLK, 1, BM)
    grid_spec = pltpu.PrefetchScalarGridSpec(
        num_scalar_prefetch=1,
        grid=(NBLK,),
        in_specs=[
            pl.BlockSpec((BM, D), lambda i, be: (i, 0)),
            pl.BlockSpec((1, FFN, D), lambda i, be: (jnp.maximum(be[i], 0), 0, 0)),
            pl.BlockSpec((1, FFN, D), lambda i, be: (jnp.maximum(be[i], 0), 0, 0)),
            pl.BlockSpec((1, D, FFN), lambda i, be: (jnp.maximum(be[i], 0), 0, 0)),
            pl.BlockSpec((1, 1, BM), lambda i, be: (i, 0, 0)),
        ],
        out_specs=pl.BlockSpec((BM, D), lambda i, be: (i, 0)),
    )
    return pl.pallas_call(
        _moe_body,
        grid_spec=grid_spec,
        out_shape=jax.ShapeDtypeStruct((PADDED, D), jnp.float32),
    )(be, xs, W1, Wg, W2, wsr)


@functools.partial(
    pl.kernel,
    out_type=jax.ShapeDtypeStruct((T, D), jnp.float32),
    mesh=_sc_mesh,
    compiler_params=_sc_params,
    name="sc_combine",
    scratch_types=[
        pltpu.VMEM((CROWS,), jnp.int32),
        pltpu.VMEM((CROWS,), jnp.int32),
        pltpu.VMEM((CCH, D), jnp.float32),
        pltpu.VMEM((CCH, D), jnp.float32),
        pltpu.VMEM((CCH, D), jnp.float32),
        pltpu.VMEM((CCH, D), jnp.float32),
        pltpu.SemaphoreType.DMA,
        pltpu.SemaphoreType.DMA,
        pltpu.SemaphoreType.DMA,
        pltpu.SemaphoreType.DMA,
    ],
)
def _sc_combine(os_hbm, pos_hbm, fin_hbm, idx0_v, idx1_v,
                a0, b0, a1, b1, sa0, sa1, sw0, sw1):
    wid = lax.axis_index("s") * NC + lax.axis_index("c")
    base = wid * CROWS
    pltpu.sync_copy(pos_hbm.at[pl.ds(base, CROWS)], idx0_v)
    pltpu.sync_copy(pos_hbm.at[pl.ds(T + base, CROWS)], idx1_v)
    nch = CROWS // CCH
    abufs, bbufs, gsems, wsems = (a0, a1), (b0, b1), (sa0, sa1), (sw0, sw1)
    getsA, getsB, puts = [None] * nch, [None] * nch, [None] * nch

    def fire(c):
        b = c % 2
        getsA[c] = pltpu.async_copy(
            os_hbm.at[idx0_v.at[pl.ds(c * CCH, CCH)]], abufs[b], gsems[b])
        getsB[c] = pltpu.async_copy(
            os_hbm.at[idx1_v.at[pl.ds(c * CCH, CCH)]], bbufs[b], gsems[b])

    fire(0)
    for c in range(nch):
        b = c % 2
        if c + 1 < nch:
            if c >= 1:
                puts[c - 1].wait()
            fire(c + 1)
        getsA[c].wait()
        getsB[c].wait()
        a_v, b_v = abufs[b], bbufs[b]

        def addrow(r, cc):
            for j in range(D // 16):
                sl = pl.ds(j * 16, 16)
                a_v[r, sl] = a_v[r, sl] + b_v[r, sl]
            return cc
        lax.fori_loop(0, CCH, addrow, 0)
        puts[c] = pltpu.async_copy(
            a_v, fin_hbm.at[pl.ds(base + c * CCH, CCH)], wsems[b])
    if nch >= 2:
        puts[nch - 2].wait()
    puts[nch - 1].wait()


@jax.jit
def kernel(hidden_states, W_gate, W1, Wg, W2):
    hs = hidden_states.reshape(T, D)
    logits, sel, wtk = _router(hs, W_gate)
    tok, ws, pos, be = _sc_sort(sel.reshape(T2), wtk.reshape(T2))
    del tok
    xs = _sc_scatter_x(hs, pos)
    os = _grouped_mlp(be, xs, W1.astype(jnp.bfloat16), Wg.astype(jnp.bfloat16),
                      W2.astype(jnp.bfloat16), ws)
    fin = _sc_combine(os, pos)
    return fin.reshape(B, S, D), logits


# speedup vs baseline: 1.6572x; 1.6572x over previous
import functools

import jax
import jax.numpy as jnp
from jax import lax
from jax.experimental import pallas as pl
from jax.experimental.pallas import tpu as pltpu
from jax.experimental.pallas import tpu_sc as plsc

B, S, D = 4, 2048, 768
E, TOP_K = 8, 2
FFN = D * 4
T = B * S
T2 = T * TOP_K

BM_R = 1024
BM = 256
PADDED = T2 + E * BM
NBLK = PADDED // BM
NBE = 80

NC, NS = 2, 16
NW = NC * NS
GROWS = PADDED // NW
GCH = 64
CROWS = T // NW
CCH = 32

_sc_mesh = plsc.VectorSubcoreMesh(core_axis_name="c", subcore_axis_name="s")


def _splat(s, dtype=jnp.int32):
    return jnp.full((16,), s, dtype)


_sc_params = pltpu.CompilerParams(needs_layout_passes=False)


def _router_body(x_ref, wg_ref, logits_ref, sel_ref, wtk_ref):
    x = x_ref[...]
    logits = lax.dot_general(x, wg_ref[...], (((1,), (1,)), ((), ())),
                             preferred_element_type=jnp.float32)
    logits_ref[...] = logits
    iota = lax.broadcasted_iota(jnp.int32, (BM_R, E), 1)
    m1 = jnp.max(logits, axis=1, keepdims=True)
    i1 = jnp.min(jnp.where(logits == m1, iota, E), axis=1, keepdims=True)
    l2 = jnp.where(iota == i1, -jnp.inf, logits)
    m2 = jnp.max(l2, axis=1, keepdims=True)
    i2 = jnp.min(jnp.where(l2 == m2, iota, E), axis=1, keepdims=True)
    w1 = 1.0 / (1.0 + jnp.exp(m2 - m1))
    sel_ref[...] = jnp.concatenate([i1, i2], axis=1)
    wtk_ref[...] = jnp.concatenate([w1, 1.0 - w1], axis=1)


def _router(hs, W_gate):
    return pl.pallas_call(
        _router_body,
        grid=(T // BM_R,),
        in_specs=[
            pl.BlockSpec((BM_R, D), lambda i: (i, 0)),
            pl.BlockSpec((E, D), lambda i: (0, 0)),
        ],
        out_specs=[
            pl.BlockSpec((BM_R, E), lambda i: (i, 0)),
            pl.BlockSpec((BM_R, TOP_K), lambda i: (i, 0)),
            pl.BlockSpec((BM_R, TOP_K), lambda i: (i, 0)),
        ],
        out_shape=[
            jax.ShapeDtypeStruct((T, E), jnp.float32),
            jax.ShapeDtypeStruct((T, TOP_K), jnp.int32),
            jax.ShapeDtypeStruct((T, TOP_K), jnp.float32),
        ],
    )(hs, W_gate)


@functools.partial(
    pl.kernel,
    out_type=[
        jax.ShapeDtypeStruct((PADDED,), jnp.int32),
        jax.ShapeDtypeStruct((PADDED,), jnp.float32),
        jax.ShapeDtypeStruct((T2,), jnp.int32),
        jax.ShapeDtypeStruct((NBE,), jnp.int32),
    ],
    mesh=_sc_mesh,
    compiler_params=_sc_params,
    name="sc_sort",
    scratch_types=[
        pltpu.VMEM((T2,), jnp.int32),
        pltpu.VMEM((T2,), jnp.float32),
        pltpu.VMEM((PADDED,), jnp.int32),
        pltpu.VMEM((PADDED,), jnp.float32),
        pltpu.VMEM((T2,), jnp.int32),
        pltpu.VMEM((NBE,), jnp.int32),
    ],
)
def _sc_sort(sel_hbm, w_hbm, tok_hbm, ws_hbm, pos_hbm, be_hbm,
             sel_v, w_v, tok_v, ws_v, pos_v, be_v):
    wid = lax.axis_index("s") * NC + lax.axis_index("c")

    @pl.when(wid == 0)
    def _():
        pltpu.sync_copy(sel_hbm, sel_v)
        pltpu.sync_copy(w_hbm, w_v)
        zi = jnp.zeros((16,), jnp.int32)
        zf = jnp.zeros((16,), jnp.float32)

        def init(i, c):
            tok_v[pl.ds(i * 16, 16)] = zi
            ws_v[pl.ds(i * 16, 16)] = zf
            return c
        lax.fori_loop(0, PADDED // 16, init, 0)

        def hist(i, cnts):
            ev = sel_v[pl.ds(i * 16, 16)]
            return tuple(cnts[e] + jnp.sum((ev == _splat(e)).astype(jnp.int32))
                         for e in range(E))
        cnts = lax.fori_loop(0, T2 // 16, hist, (jnp.int32(0),) * E)

        starts, ends = [], []
        s = jnp.int32(0)
        for e in range(E):
            starts.append(s)
            s = s + ((cnts[e] + BM - 1) // BM) * BM
            ends.append(s)
        total = s

        iota16 = lax.iota(jnp.int32, 16)
        for vb in range(NBE // 16):
            bstart = (iota16 + _splat(vb * 16)) * _splat(BM)
            acc = jnp.zeros((16,), jnp.int32)
            for e in range(E):
                acc = acc + (bstart >= _splat(ends[e])).astype(jnp.int32)
            be_v[pl.ds(vb * 16, 16)] = jnp.where(bstart < _splat(total), acc,
                                                 _splat(-1))

        def scat(i, offs):
            ds = pl.ds(i * 16, 16)
            jv = _splat(i * 16) + iota16
            ev = sel_v[ds]
            wv = w_v[ds]
            tv = jv >> _splat(1)
            kv = jv & _splat(1)
            posidx = kv * _splat(T) + tv
            p = jnp.zeros((16,), jnp.int32)
            new_offs = []
            for e in range(E):
                me = ev == _splat(e)
                mi = me.astype(jnp.int32)
                cs = jnp.cumsum(mi)
                p = jnp.where(me, _splat(offs[e]) + cs - _splat(1), p)
                new_offs.append(offs[e] + jnp.sum(mi))
            plsc.store_scatter(tok_v, [p], tv)
            plsc.store_scatter(ws_v, [p], wv)
            plsc.store_scatter(pos_v, [posidx], p)
            return tuple(new_offs)
        lax.fori_loop(0, T2 // 16, scat, tuple(starts))

        pltpu.sync_copy(tok_v, tok_hbm)
        pltpu.sync_copy(ws_v, ws_hbm)
        pltpu.sync_copy(pos_v, pos_hbm)
        pltpu.sync_copy(be_v, be_hbm)


GCH2 = 32


@functools.partial(
    pl.kernel,
    out_type=jax.ShapeDtypeStruct((PADDED, D), jnp.float32),
    mesh=_sc_mesh,
    compiler_params=_sc_params,
    name="sc_scatter_x",
    scratch_types=[
        pltpu.VMEM((GCH2,), jnp.int32),
        pltpu.VMEM((GCH2,), jnp.int32),
        pltpu.VMEM((GCH2,), jnp.int32),
        pltpu.VMEM((GCH2,), jnp.int32),
        pltpu.VMEM((GCH2, D), jnp.float32),
        pltpu.VMEM((GCH2, D), jnp.float32),
        pltpu.SemaphoreType.DMA,
        pltpu.SemaphoreType.DMA,
        pltpu.SemaphoreType.DMA,
        pltpu.SemaphoreType.DMA,
    ],
)
def _sc_scatter_x(hs_hbm, pos_hbm, xs_hbm,
                  i0a, i1a, i0b, i1b, rows0, rows1, sr0, sr1, sw0, sw1):
    wid = lax.axis_index("s") * NC + lax.axis_index("c")
    base = wid * CROWS
    nch = CROWS // GCH2
    bufs, i0s, i1s = (rows0, rows1), (i0a, i0b), (i1a, i1b)
    rsems, wsems = (sr0, sr1), (sw0, sw1)
    gets, putsA, putsB = [None] * nch, [None] * nch, [None] * nch

    def fire(c):
        gets[c] = pltpu.async_copy(
            hs_hbm.at[pl.ds(base + c * GCH2, GCH2)], bufs[c % 2], rsems[c % 2])

    fire(0)
    for c in range(nch):
        b = c % 2
        if c + 1 < nch:
            if c >= 1:
                putsA[c - 1].wait()
                putsB[c - 1].wait()
            fire(c + 1)
        off = base + c * GCH2
        pltpu.sync_copy(pos_hbm.at[pl.ds(off, GCH2)], i0s[b])
        pltpu.sync_copy(pos_hbm.at[pl.ds(T + off, GCH2)], i1s[b])
        gets[c].wait()
        putsA[c] = pltpu.async_copy(bufs[b], xs_hbm.at[i0s[b]], wsems[b])
        putsB[c] = pltpu.async_copy(bufs[b], xs_hbm.at[i1s[b]], wsems[b])
    if nch >= 2:
        putsA[nch - 2].wait()
        putsB[nch - 2].wait()
    putsA[nch - 1].wait()
    putsB[nch - 1].wait()


def _moe_body(be_ref, x_ref, w1_ref, wg_ref, w2_ref, wt_ref, out_ref):
    i = pl.program_id(0)

    @pl.when(be_ref[i] >= 0)
    def _():
        x = x_ref[...].astype(jnp.bfloat16)
        a = lax.dot_general(x, w1_ref[0], (((1,), (1,)), ((), ())),
                            preferred_element_type=jnp.float32)
        g = lax.dot_general(x, wg_ref[0], (((1,), (1,)), ((), ())),
                            preferred_element_type=jnp.float32)
        h = ((a * jax.nn.sigmoid(a)) * g).astype(jnp.bfloat16)
        o = lax.dot_general(h, w2_ref[0], (((1,), (1,)), ((), ())),
                            preferred_element_type=jnp.float32)
        out_ref[...] = o * wt_ref[0, 0][:, None]


def _grouped_mlp(be, xs, W1, Wg, W2, ws):
    wsr = ws.reshape(NBLK, 1, BM)
    grid_spec = pltpu.PrefetchScalarGridSpec(
        num_scalar_prefetch=1,
        grid=(NBLK,),
        in_specs=[
            pl.BlockSpec((BM, D), lambda i, be: (i, 0)),
            pl.BlockSpec((1, FFN, D), lambda i, be: (jnp.maximum(be[i], 0), 0, 0)),
            pl.BlockSpec((1, FFN, D), lambda i, be: (jnp.maximum(be[i], 0), 0, 0)),
            pl.BlockSpec((1, D, FFN), lambda i, be: (jnp.maximum(be[i], 0), 0, 0)),
            pl.BlockSpec((1, 1, BM), lambda i, be: (i, 0, 0)),
        ],
        out_specs=pl.BlockSpec((BM, D), lambda i, be: (i, 0)),
    )
    return pl.pallas_call(
        _moe_body,
        grid_spec=grid_spec,
        out_shape=jax.ShapeDtypeStruct((PADDED, D), jnp.float32),
    )(be, xs, W1, Wg, W2, wsr)


@functools.partial(
    pl.kernel,
    out_type=jax.ShapeDtypeStruct((T, D), jnp.float32),
    mesh=_sc_mesh,
    compiler_params=_sc_params,
    name="sc_combine",
    scratch_types=[
        pltpu.VMEM((CROWS,), jnp.int32),
        pltpu.VMEM((CROWS,), jnp.int32),
        pltpu.VMEM((CCH, D), jnp.float32),
        pltpu.VMEM((CCH, D), jnp.float32),
        pltpu.VMEM((CCH, D), jnp.float32),
        pltpu.VMEM((CCH, D), jnp.float32),
        pltpu.SemaphoreType.DMA,
        pltpu.SemaphoreType.DMA,
        pltpu.SemaphoreType.DMA,
        pltpu.SemaphoreType.DMA,
    ],
)
def _sc_combine(os_hbm, pos_hbm, fin_hbm, idx0_v, idx1_v,
                a0, b0, a1, b1, sa0, sa1, sw0, sw1):
    wid = lax.axis_index("s") * NC + lax.axis_index("c")
    base = wid * CROWS
    pltpu.sync_copy(pos_hbm.at[pl.ds(base, CROWS)], idx0_v)
    pltpu.sync_copy(pos_hbm.at[pl.ds(T + base, CROWS)], idx1_v)
    nch = CROWS // CCH
    abufs, bbufs, gsems, wsems = (a0, a1), (b0, b1), (sa0, sa1), (sw0, sw1)
    getsA, getsB, puts = [None] * nch, [None] * nch, [None] * nch

    def fire(c):
        b = c % 2
        getsA[c] = pltpu.async_copy(
            os_hbm.at[idx0_v.at[pl.ds(c * CCH, CCH)]], abufs[b], gsems[b])
        getsB[c] = pltpu.async_copy(
            os_hbm.at[idx1_v.at[pl.ds(c * CCH, CCH)]], bbufs[b], gsems[b])

    fire(0)
    for c in range(nch):
        b = c % 2
        if c + 1 < nch:
            if c >= 1:
                puts[c - 1].wait()
            fire(c + 1)
        getsA[c].wait()
        getsB[c].wait()
        a_v, b_v = abufs[b], bbufs[b]

        def addrow(r, cc):
            for j in range(D // 16):
                sl = pl.ds(j * 16, 16)
                a_v[r, sl] = a_v[r, sl] + b_v[r, sl]
            return cc
        lax.fori_loop(0, CCH, addrow, 0)
        puts[c] = pltpu.async_copy(
            a_v, fin_hbm.at[pl.ds(base + c * CCH, CCH)], wsems[b])
    if nch >= 2:
        puts[nch - 2].wait()
    puts[nch - 1].wait()


@jax.jit
def kernel(hidden_states, W_gate, W1, Wg, W2):
    hs = hidden_states.reshape(T, D)
    logits, sel, wtk = _router(hs, W_gate)
    tok, ws, pos, be = _sc_sort(sel.reshape(T2), wtk.reshape(T2))
    del tok
    xs = _sc_scatter_x(hs, pos)
    os = _grouped_mlp(be, xs, W1.astype(jnp.bfloat16), Wg.astype(jnp.bfloat16),
                      W2.astype(jnp.bfloat16), ws)
    fin = _sc_combine(os, pos)
    return fin.reshape(B, S, D), logits


# speedup vs baseline: 1.6594x; 1.0013x over previous
import functools

import jax
import jax.numpy as jnp
from jax import lax
from jax.experimental import pallas as pl
from jax.experimental.pallas import tpu as pltpu
from jax.experimental.pallas import tpu_sc as plsc

B, S, D = 4, 2048, 768
E, TOP_K = 8, 2
FFN = D * 4
T = B * S
T2 = T * TOP_K

BM_R = 1024
BM = 256
PADDED = T2 + E * BM
NBLK = PADDED // BM
NBE = 80

NC, NS = 2, 16
NW = NC * NS
GROWS = PADDED // NW
GCH = 64
CROWS = T // NW
CCH = 32

_sc_mesh = plsc.VectorSubcoreMesh(core_axis_name="c", subcore_axis_name="s")


def _splat(s, dtype=jnp.int32):
    return jnp.full((16,), s, dtype)


_sc_params = pltpu.CompilerParams(needs_layout_passes=False)


def _router_body(x_ref, wg_ref, logits_ref, sel_ref, wtk_ref):
    x = x_ref[...]
    logits = lax.dot_general(x, wg_ref[...], (((1,), (1,)), ((), ())),
                             preferred_element_type=jnp.float32)
    logits_ref[...] = logits
    iota = lax.broadcasted_iota(jnp.int32, (BM_R, E), 1)
    m1 = jnp.max(logits, axis=1, keepdims=True)
    i1 = jnp.min(jnp.where(logits == m1, iota, E), axis=1, keepdims=True)
    l2 = jnp.where(iota == i1, -jnp.inf, logits)
    m2 = jnp.max(l2, axis=1, keepdims=True)
    i2 = jnp.min(jnp.where(l2 == m2, iota, E), axis=1, keepdims=True)
    w1 = 1.0 / (1.0 + jnp.exp(m2 - m1))
    sel_ref[...] = jnp.concatenate([i1, i2], axis=1)
    wtk_ref[...] = jnp.concatenate([w1, 1.0 - w1], axis=1)


def _router(hs, W_gate):
    return pl.pallas_call(
        _router_body,
        grid=(T // BM_R,),
        in_specs=[
            pl.BlockSpec((BM_R, D), lambda i: (i, 0)),
            pl.BlockSpec((E, D), lambda i: (0, 0)),
        ],
        out_specs=[
            pl.BlockSpec((BM_R, E), lambda i: (i, 0)),
            pl.BlockSpec((BM_R, TOP_K), lambda i: (i, 0)),
            pl.BlockSpec((BM_R, TOP_K), lambda i: (i, 0)),
        ],
        out_shape=[
            jax.ShapeDtypeStruct((T, E), jnp.float32),
            jax.ShapeDtypeStruct((T, TOP_K), jnp.int32),
            jax.ShapeDtypeStruct((T, TOP_K), jnp.float32),
        ],
    )(hs, W_gate)


@functools.partial(
    pl.kernel,
    out_type=[
        jax.ShapeDtypeStruct((PADDED,), jnp.int32),
        jax.ShapeDtypeStruct((PADDED,), jnp.float32),
        jax.ShapeDtypeStruct((T2,), jnp.int32),
        jax.ShapeDtypeStruct((NBE,), jnp.int32),
    ],
    mesh=_sc_mesh,
    compiler_params=_sc_params,
    name="sc_sort",
    scratch_types=[
        pltpu.VMEM((T2,), jnp.int32),
        pltpu.VMEM((T2,), jnp.float32),
        pltpu.VMEM((PADDED,), jnp.int32),
        pltpu.VMEM((PADDED,), jnp.float32),
        pltpu.VMEM((T2,), jnp.int32),
        pltpu.VMEM((NBE,), jnp.int32),
    ],
)
def _sc_sort(sel_hbm, w_hbm, tok_hbm, ws_hbm, pos_hbm, be_hbm,
             sel_v, w_v, tok_v, ws_v, pos_v, be_v):
    wid = lax.axis_index("s") * NC + lax.axis_index("c")

    @pl.when(wid == 0)
    def _():
        pltpu.sync_copy(sel_hbm, sel_v)
        pltpu.sync_copy(w_hbm, w_v)
        zi = jnp.zeros((16,), jnp.int32)
        zf = jnp.zeros((16,), jnp.float32)

        def init(i, c):
            tok_v[pl.ds(i * 16, 16)] = zi
            ws_v[pl.ds(i * 16, 16)] = zf
            return c
        lax.fori_loop(0, PADDED // 16, init, 0)

        def hist(i, cnts):
            ev = sel_v[pl.ds(i * 16, 16)]
            return tuple(cnts[e] + jnp.sum((ev == _splat(e)).astype(jnp.int32))
                         for e in range(E))
        cnts = lax.fori_loop(0, T2 // 16, hist, (jnp.int32(0),) * E)

        starts, ends = [], []
        s = jnp.int32(0)
        for e in range(E):
            starts.append(s)
            s = s + ((cnts[e] + BM - 1) // BM) * BM
            ends.append(s)
        total = s

        iota16 = lax.iota(jnp.int32, 16)
        for vb in range(NBE // 16):
            bstart = (iota16 + _splat(vb * 16)) * _splat(BM)
            acc = jnp.zeros((16,), jnp.int32)
            for e in range(E):
                acc = acc + (bstart >= _splat(ends[e])).astype(jnp.int32)
            be_v[pl.ds(vb * 16, 16)] = jnp.where(bstart < _splat(total), acc,
                                                 _splat(-1))

        def scat(i, offs):
            ds = pl.ds(i * 16, 16)
            jv = _splat(i * 16) + iota16
            ev = sel_v[ds]
            wv = w_v[ds]
            tv = jv >> _splat(1)
            kv = jv & _splat(1)
            posidx = kv * _splat(T) + tv
            p = jnp.zeros((16,), jnp.int32)
            new_offs = []
            for e in range(E):
                me = ev == _splat(e)
                mi = me.astype(jnp.int32)
                cs = jnp.cumsum(mi)
                p = jnp.where(me, _splat(offs[e]) + cs - _splat(1), p)
                new_offs.append(offs[e] + jnp.sum(mi))
            plsc.store_scatter(tok_v, [p], tv)
            plsc.store_scatter(ws_v, [p], wv)
            plsc.store_scatter(pos_v, [posidx], p)
            return tuple(new_offs)
        lax.fori_loop(0, T2 // 16, scat, tuple(starts))

        pltpu.sync_copy(tok_v, tok_hbm)
        pltpu.sync_copy(ws_v, ws_hbm)
        pltpu.sync_copy(pos_v, pos_hbm)
        pltpu.sync_copy(be_v, be_hbm)


GCH2 = 32


@functools.partial(
    pl.kernel,
    out_type=jax.ShapeDtypeStruct((PADDED, D), jnp.float32),
    mesh=_sc_mesh,
    compiler_params=_sc_params,
    name="sc_scatter_x",
    scratch_types=[
        pltpu.VMEM((GCH2,), jnp.int32),
        pltpu.VMEM((GCH2,), jnp.int32),
        pltpu.VMEM((GCH2,), jnp.int32),
        pltpu.VMEM((GCH2,), jnp.int32),
        pltpu.VMEM((GCH2, D), jnp.float32),
        pltpu.VMEM((GCH2, D), jnp.float32),
        pltpu.SemaphoreType.DMA,
        pltpu.SemaphoreType.DMA,
        pltpu.SemaphoreType.DMA,
        pltpu.SemaphoreType.DMA,
    ],
)
def _sc_scatter_x(hs_hbm, pos_hbm, xs_hbm,
                  i0a, i1a, i0b, i1b, rows0, rows1, sr0, sr1, sw0, sw1):
    wid = lax.axis_index("s") * NC + lax.axis_index("c")
    base = wid * CROWS
    nch = CROWS // GCH2
    bufs, i0s, i1s = (rows0, rows1), (i0a, i0b), (i1a, i1b)
    rsems, wsems = (sr0, sr1), (sw0, sw1)
    gets, putsA, putsB = [None] * nch, [None] * nch, [None] * nch

    def fire(c):
        gets[c] = pltpu.async_copy(
            hs_hbm.at[pl.ds(base + c * GCH2, GCH2)], bufs[c % 2], rsems[c % 2])

    fire(0)
    for c in range(nch):
        b = c % 2
        if c + 1 < nch:
            if c >= 1:
                putsA[c - 1].wait()
                putsB[c - 1].wait()
            fire(c + 1)
        off = base + c * GCH2
        pltpu.sync_copy(pos_hbm.at[pl.ds(off, GCH2)], i0s[b])
        pltpu.sync_copy(pos_hbm.at[pl.ds(T + off, GCH2)], i1s[b])
        gets[c].wait()
        putsA[c] = pltpu.async_copy(bufs[b], xs_hbm.at[i0s[b]], wsems[b])
        putsB[c] = pltpu.async_copy(bufs[b], xs_hbm.at[i1s[b]], wsems[b])
    if nch >= 2:
        putsA[nch - 2].wait()
        putsB[nch - 2].wait()
    putsA[nch - 1].wait()
    putsB[nch - 1].wait()


def _moe_body(be_ref, x_ref, w1_ref, wg_ref, w2_ref, wt_ref, out_ref):
    i = pl.program_id(0)

    @pl.when(be_ref[i] >= 0)
    def _():
        x = x_ref[...].astype(jnp.bfloat16)
        a = lax.dot_general(x, w1_ref[0], (((1,), (1,)), ((), ())),
                            preferred_element_type=jnp.float32)
        g = lax.dot_general(x, wg_ref[0], (((1,), (1,)), ((), ())),
                            preferred_element_type=jnp.float32)
        h = ((a * jax.nn.sigmoid(a)) * g).astype(jnp.bfloat16)
        o = lax.dot_general(h, w2_ref[0], (((1,), (1,)), ((), ())),
                            preferred_element_type=jnp.float32)
        out_ref[...] = o * wt_ref[0, 0][:, None]


def _grouped_mlp(be, xs, W1, Wg, W2, ws):
    wsr = ws.reshape(NBLK, 1, BM)
    grid_spec = pltpu.PrefetchScalarGridSpec(
        num_scalar_prefetch=1,
        grid=(NBLK,),
        in_specs=[
            pl.BlockSpec((BM, D), lambda i, be: (i, 0)),
            pl.BlockSpec((1, FFN, D), lambda i, be: (jnp.maximum(be[i], 0), 0, 0)),
            pl.BlockSpec((1, FFN, D), lambda i, be: (jnp.maximum(be[i], 0), 0, 0)),
            pl.BlockSpec((1, D, FFN), lambda i, be: (jnp.maximum(be[i], 0), 0, 0)),
            pl.BlockSpec((1, 1, BM), lambda i, be: (i, 0, 0)),
        ],
        out_specs=pl.BlockSpec((BM, D), lambda i, be: (i, 0)),
    )
    return pl.pallas_call(
        _moe_body,
        grid_spec=grid_spec,
        out_shape=jax.ShapeDtypeStruct((PADDED, D), jnp.float32),
    )(be, xs, W1, Wg, W2, wsr)


@functools.partial(
    pl.kernel,
    out_type=jax.ShapeDtypeStruct((T, D), jnp.float32),
    mesh=_sc_mesh,
    compiler_params=_sc_params,
    name="sc_combine",
    scratch_types=[
        pltpu.VMEM((CROWS,), jnp.int32),
        pltpu.VMEM((CROWS,), jnp.int32),
        pltpu.VMEM((CCH, D), jnp.float32),
        pltpu.VMEM((CCH, D), jnp.float32),
        pltpu.VMEM((CCH, D), jnp.float32),
        pltpu.VMEM((CCH, D), jnp.float32),
        pltpu.SemaphoreType.DMA,
        pltpu.SemaphoreType.DMA,
        pltpu.SemaphoreType.DMA,
        pltpu.SemaphoreType.DMA,
    ],
)
def _sc_combine(os_hbm, pos_hbm, fin_hbm, idx0_v, idx1_v,
                a0, b0, a1, b1, sa0, sa1, sw0, sw1):
    wid = lax.axis_index("s") * NC + lax.axis_index("c")
    base = wid * CROWS
    pltpu.sync_copy(pos_hbm.at[pl.ds(base, CROWS)], idx0_v)
    pltpu.sync_copy(pos_hbm.at[pl.ds(T + base, CROWS)], idx1_v)
    nch = CROWS // CCH
    abufs, bbufs, gsems, wsems = (a0, a1), (b0, b1), (sa0, sa1), (sw0, sw1)
    getsA, getsB, puts = [None] * nch, [None] * nch, [None] * nch

    def fire(c):
        b = c % 2
        getsA[c] = pltpu.async_copy(
            os_hbm.at[idx0_v.at[pl.ds(c * CCH, CCH)]], abufs[b], gsems[b])
        getsB[c] = pltpu.async_copy(
            os_hbm.at[idx1_v.at[pl.ds(c * CCH, CCH)]], bbufs[b], gsems[b])

    fire(0)
    for c in range(nch):
        b = c % 2
        if c + 1 < nch:
            if c >= 1:
                puts[c - 1].wait()
            fire(c + 1)
        getsA[c].wait()
        getsB[c].wait()
        a_v, b_v = abufs[b], bbufs[b]

        def addrow(r, cc):
            for j in range(D // 16):
                sl = pl.ds(j * 16, 16)
                a_v[r, sl] = a_v[r, sl] + b_v[r, sl]
            return cc
        lax.fori_loop(0, CCH, addrow, 0)
        puts[c] = pltpu.async_copy(
            a_v, fin_hbm.at[pl.ds(base + c * CCH, CCH)], wsems[b])
    if nch >= 2:
        puts[nch - 2].wait()
    puts[nch - 1].wait()


@jax.jit
def kernel(hidden_states, W_gate, W1, Wg, W2):
    hs = hidden_states.reshape(T, D)
    W1b = W1.astype(jnp.bfloat16)
    Wgb = Wg.astype(jnp.bfloat16)
    W2b = W2.astype(jnp.bfloat16)
    logits, sel, wtk = _router(hs, W_gate)
    tok, ws, pos, be = _sc_sort(sel.reshape(T2), wtk.reshape(T2))
    del tok
    xs = _sc_scatter_x(hs, pos)
    os = _grouped_mlp(be, xs, W1b, Wgb, W2b, ws)
    fin = _sc_combine(os, pos)
    return fin.reshape(B, S, D), logits
